# baseline (device time: 1411754 ns/iter reference)
import jax
import jax.numpy as jnp
from jax import lax
from jax.experimental import pallas as pl
from jax.experimental.pallas import tpu as pltpu

N_DEV = 32
E_LOC = 4
E_TOT = 128
TOK = 1024
D = 512
H = 1024
RC = 128
CW = D + H + RC


def kernel(x, router_W, route_idx, expert_W, shared_W):
    def body(x_ref, rW_ref, idx_ref, eW_ref, sW_ref, out_ref,
             comm_ref, eWbf_ref, send_sems, recv_sems, credit_sem):
        my = lax.axis_index("i")
        left = lax.rem(my + N_DEV - 1, N_DEV)
        right = lax.rem(my + 1, N_DEV)

        barrier = pltpu.get_barrier_semaphore()
        pl.semaphore_signal(barrier, inc=1, device_id=(left,),
                            device_id_type=pl.DeviceIdType.MESH)
        pl.semaphore_signal(barrier, inc=1, device_id=(right,),
                            device_id_type=pl.DeviceIdType.MESH)
        pl.semaphore_wait(barrier, 2)

        x32 = x_ref[:, :]
        scores = jnp.dot(x32, rW_ref[:, :], preferred_element_type=jnp.float32)
        smax = jnp.max(scores, axis=-1, keepdims=True)
        p = jnp.exp(scores - smax)
        p = p / jnp.sum(p, axis=-1, keepdims=True)
        rt = idx_ref[:, :]
        onehot = lax.broadcasted_iota(jnp.int32, (TOK, E_TOT), 1) == rt
        selp = jnp.sum(jnp.where(onehot, p, 0.0), axis=1, keepdims=True)

        xbf = x32.astype(jnp.bfloat16)
        out_ref[:, :] = jnp.dot(xbf, sW_ref[:, :].astype(jnp.bfloat16),
                                preferred_element_type=jnp.float32)

        eWbf_ref[:, :] = eW_ref[:, :, :].astype(jnp.bfloat16).reshape(E_LOC * D, H)

        comm_ref[0, :, :D] = (x32 * selp).astype(jnp.bfloat16)
        comm_ref[0, :, D:D + H] = jnp.zeros((TOK, H), jnp.bfloat16)
        comm_ref[0, :, D + H:] = jnp.broadcast_to(
            rt.astype(jnp.bfloat16), (TOK, RC))

        for h in range(N_DEV):
            slot, nxt = h % 2, (h + 1) % 2
            xin = comm_ref[slot, :, :D]
            rts = comm_ref[slot, :, D + H:D + H + 1]
            parts = []
            for k in range(E_LOC):
                eid = (my * E_LOC + k).astype(jnp.bfloat16)
                parts.append(jnp.where(rts == eid, xin, jnp.bfloat16(0.0)))
            xcat = jnp.concatenate(parts, axis=1)
            contrib = jnp.dot(xcat, eWbf_ref[:, :],
                              preferred_element_type=jnp.float32)
            acc = comm_ref[slot, :, D:D + H].astype(jnp.float32) + contrib
            comm_ref[slot, :, D:D + H] = acc.astype(jnp.bfloat16)

            if h > 0:
                pl.semaphore_wait(credit_sem, 1)
            rdma = pltpu.make_async_remote_copy(
                src_ref=comm_ref.at[slot],
                dst_ref=comm_ref.at[nxt],
                send_sem=send_sems.at[slot],
                recv_sem=recv_sems.at[nxt],
                device_id=(right,),
                device_id_type=pl.DeviceIdType.MESH,
            )
            rdma.start()
            rdma.wait()
            if h < N_DEV - 1:
                pl.semaphore_signal(credit_sem, inc=1, device_id=(left,),
                                    device_id_type=pl.DeviceIdType.MESH)

        out_ref[:, :] = out_ref[:, :] + comm_ref[0, :, D:D + H].astype(jnp.float32)

    return pl.pallas_call(
        body,
        out_shape=jax.ShapeDtypeStruct((TOK, H), jnp.float32),
        in_specs=[pl.BlockSpec(memory_space=pltpu.VMEM)] * 5,
        out_specs=pl.BlockSpec(memory_space=pltpu.VMEM),
        scratch_shapes=[
            pltpu.VMEM((2, TOK, CW), jnp.bfloat16),
            pltpu.VMEM((E_LOC * D, H), jnp.bfloat16),
            pltpu.SemaphoreType.DMA((2,)),
            pltpu.SemaphoreType.DMA((2,)),
            pltpu.SemaphoreType.REGULAR,
        ],
        compiler_params=pltpu.CompilerParams(collective_id=0),
    )(x, router_W, route_idx, expert_W, shared_W)


# device time: 186503 ns/iter; 7.5696x vs baseline; 7.5696x over previous
import jax
import jax.numpy as jnp
from jax import lax
from jax.experimental import pallas as pl
from jax.experimental.pallas import tpu as pltpu

N_DEV = 32
E_LOC = 4
E_TOT = 128
TOK = 1024
D = 512
H = 1024
C = 64
PW = D + 128


def kernel(x, router_W, route_idx, expert_W, shared_W):
    scores = jnp.dot(x, router_W, preferred_element_type=jnp.float32)
    p = jax.nn.softmax(scores, axis=-1)
    selp = jnp.take_along_axis(p, route_idx, axis=1)
    route = route_idx[:, 0]
    dst = route // E_LOC
    eloc = route % E_LOC

    order = jnp.argsort(route)
    dst_s = dst[order]
    counts = jnp.zeros((N_DEV,), jnp.int32).at[dst].add(1)
    start = jnp.cumsum(counts) - counts
    pos = jnp.minimum(jnp.arange(TOK, dtype=jnp.int32) - start[dst_s], C - 1)
    slot_s = dst_s * C + pos

    xs = (x * selp).astype(jnp.bfloat16)[order]
    eloc_s = eloc[order].astype(jnp.bfloat16)
    payload = jnp.concatenate(
        [xs, jnp.broadcast_to(eloc_s[:, None], (TOK, 128))], axis=1)
    sendbuf = (jnp.zeros((N_DEV * C, PW), jnp.bfloat16)
               .at[slot_s].set(payload).reshape(N_DEV, C, PW))
    slot_per_token = jnp.zeros((TOK,), jnp.int32).at[order].set(slot_s)

    def body(x_ref, sW_ref, eW_ref, send_ref, outs_ref, res_ref,
             recvbuf, resbuf, ssem1, rsem1, ssem2, rsem2):
        my = lax.axis_index("i")

        barrier = pltpu.get_barrier_semaphore()
        for j in range(1, N_DEV):
            pl.semaphore_signal(barrier, inc=1,
                                device_id=(lax.rem(my + j, N_DEV),),
                                device_id_type=pl.DeviceIdType.MESH)
        pl.semaphore_wait(barrier, N_DEV - 1)

        disp = []
        for j in range(N_DEV):
            t = lax.rem(my + j, N_DEV)
            rdma = pltpu.make_async_remote_copy(
                src_ref=send_ref.at[t],
                dst_ref=recvbuf.at[my],
                send_sem=ssem1.at[j],
                recv_sem=rsem1.at[j],
                device_id=(t,),
                device_id_type=pl.DeviceIdType.MESH,
            )
            rdma.start()
            disp.append(rdma)

        xbf = x_ref[:, :].astype(jnp.bfloat16)
        outs_ref[:, :] = jnp.dot(xbf, sW_ref[:, :].astype(jnp.bfloat16),
                                 preferred_element_type=jnp.float32)

        for j in range(N_DEV):
            disp[j].wait()

        val = recvbuf[:, :, :]
        xin = val[:, :, :D].reshape(N_DEV * C, D)
        el = val[:, :, D:D + 1].reshape(N_DEV * C, 1)
        contrib = jnp.zeros((N_DEV * C, H), jnp.float32)
        for k in range(E_LOC):
            xm = jnp.where(el == jnp.bfloat16(k), xin, jnp.bfloat16(0.0))
            contrib = contrib + jnp.dot(
                xm, eW_ref[k, :, :].astype(jnp.bfloat16),
                preferred_element_type=jnp.float32)
        resbuf[:, :, :] = contrib.astype(jnp.bfloat16).reshape(N_DEV, C, H)

        comb = []
        for j in range(N_DEV):
            t = lax.rem(my + j, N_DEV)
            rdma = pltpu.make_async_remote_copy(
                src_ref=resbuf.at[t],
                dst_ref=res_ref.at[my],
                send_sem=ssem2.at[j],
                recv_sem=rsem2.at[j],
                device_id=(t,),
                device_id_type=pl.DeviceIdType.MESH,
            )
            rdma.start()
            comb.append(rdma)
        for j in range(N_DEV):
            comb[j].wait()

    outs, res = pl.pallas_call(
        body,
        out_shape=(
            jax.ShapeDtypeStruct((TOK, H), jnp.float32),
            jax.ShapeDtypeStruct((N_DEV, C, H), jnp.bfloat16),
        ),
        in_specs=[pl.BlockSpec(memory_space=pltpu.VMEM)] * 4,
        out_specs=(pl.BlockSpec(memory_space=pltpu.VMEM),
                   pl.BlockSpec(memory_space=pltpu.VMEM)),
        scratch_shapes=[
            pltpu.VMEM((N_DEV, C, PW), jnp.bfloat16),
            pltpu.VMEM((N_DEV, C, H), jnp.bfloat16),
            pltpu.SemaphoreType.DMA((N_DEV,)),
            pltpu.SemaphoreType.DMA((N_DEV,)),
            pltpu.SemaphoreType.DMA((N_DEV,)),
            pltpu.SemaphoreType.DMA((N_DEV,)),
        ],
        compiler_params=pltpu.CompilerParams(collective_id=0),
    )(x, shared_W, expert_W, sendbuf)

    expert_out = res.reshape(N_DEV * C, H)[slot_per_token].astype(jnp.float32)
    return outs + expert_out


# device time: 131191 ns/iter; 10.7611x vs baseline; 1.4216x over previous
import jax
import jax.numpy as jnp
from jax import lax
from jax.experimental import pallas as pl
from jax.experimental.pallas import tpu as pltpu

N_DEV = 32
E_LOC = 4
E_TOT = 128
TOK = 1024
D = 512
H = 1024
C = 64
PW = D + 128
S = N_DEV * C


def kernel(x, router_W, route_idx, expert_W, shared_W):
    route_row = route_idx.reshape(1, TOK)

    def body(x_ref, rW_ref, idx_ref, idxr_ref, eW_ref, sW_ref, outs_ref,
             sendbuf, recvbuf, resbuf, resrec, ssem1, rsem1, ssem2, rsem2):
        my = lax.axis_index("i")

        barrier = pltpu.get_barrier_semaphore()
        for j in range(1, N_DEV):
            pl.semaphore_signal(barrier, inc=1,
                                device_id=(lax.rem(my + j, N_DEV),),
                                device_id_type=pl.DeviceIdType.MESH,)
        pl.semaphore_wait(barrier, N_DEV - 1)

        x32 = x_ref[:, :]
        scores = jnp.dot(x32, rW_ref[:, :], preferred_element_type=jnp.float32)
        smax = jnp.max(scores, axis=-1, keepdims=True)
        pr = jnp.exp(scores - smax)
        pr = pr / jnp.sum(pr, axis=-1, keepdims=True)
        rt_c = idx_ref[:, :]
        rt_r = idxr_ref[:, :]
        onehot = lax.broadcasted_iota(jnp.int32, (TOK, E_TOT), 1) == rt_c
        selp = jnp.sum(jnp.where(onehot, pr, 0.0), axis=1, keepdims=True)
        xs = (x32 * selp).astype(jnp.bfloat16)

        dst_c, dst_r = rt_c // E_LOC, rt_r // E_LOC
        eloc_c = (rt_c % E_LOC).astype(jnp.bfloat16)
        i0 = lax.broadcasted_iota(jnp.int32, (TOK, TOK), 0)
        i1 = lax.broadcasted_iota(jnp.int32, (TOK, TOK), 1)
        same = (dst_c == dst_r)
        pos_c = jnp.sum((same & (i1 < i0)).astype(jnp.int32),
                        axis=1, keepdims=True)
        pos_r = jnp.sum((same & (i0 < i1)).astype(jnp.int32),
                        axis=0, keepdims=True)
        slot_c = dst_c * C + jnp.minimum(pos_c, C - 1)
        slot_r = dst_r * C + jnp.minimum(pos_r, C - 1)

        payload = jnp.concatenate(
            [xs, jnp.broadcast_to(eloc_c, (TOK, 128))], axis=1)
        s2 = (lax.broadcasted_iota(jnp.int32, (S, TOK), 0)
              == slot_r).astype(jnp.bfloat16)
        sendbuf[:, :, :] = jnp.dot(
            s2, payload, preferred_element_type=jnp.float32
        ).astype(jnp.bfloat16).reshape(N_DEV, C, PW)

        disp = []
        for j in range(N_DEV):
            t = lax.rem(my + j, N_DEV)
            rdma = pltpu.make_async_remote_copy(
                src_ref=sendbuf.at[t],
                dst_ref=recvbuf.at[my],
                send_sem=ssem1.at[j],
                recv_sem=rsem1.at[j],
                device_id=(t,),
                device_id_type=pl.DeviceIdType.MESH,
            )
            rdma.start()
            disp.append(rdma)

        outs_ref[:, :] = jnp.dot(
            x32.astype(jnp.bfloat16), sW_ref[:, :],
            preferred_element_type=jnp.float32)

        for j in range(N_DEV):
            disp[j].wait()

        val = recvbuf[:, :, :]
        xin = val[:, :, :D].reshape(S, D)
        el = val[:, :, D:D + 1].reshape(S, 1)
        contrib = jnp.zeros((S, H), jnp.float32)
        for k in range(E_LOC):
            xm = jnp.where(el == jnp.bfloat16(k), xin, jnp.bfloat16(0.0))
            contrib = contrib + jnp.dot(xm, eW_ref[k, :, :],
                                        preferred_element_type=jnp.float32)
        resbuf[:, :, :] = contrib.astype(jnp.bfloat16).reshape(N_DEV, C, H)

        comb = []
        for j in range(N_DEV):
            t = lax.rem(my + j, N_DEV)
            rdma = pltpu.make_async_remote_copy(
                src_ref=resbuf.at[t],
                dst_ref=resrec.at[my],
                send_sem=ssem2.at[j],
                recv_sem=rsem2.at[j],
                device_id=(t,),
                device_id_type=pl.DeviceIdType.MESH,
            )
            rdma.start()
            comb.append(rdma)
        for j in range(N_DEV):
            comb[j].wait()

        p_g = (slot_c == lax.broadcasted_iota(jnp.int32, (TOK, S), 1)
               ).astype(jnp.bfloat16)
        expert_out = jnp.dot(p_g, resrec[:, :, :].reshape(S, H),
                             preferred_element_type=jnp.float32)
        outs_ref[:, :] = outs_ref[:, :] + expert_out

    return pl.pallas_call(
        body,
        out_shape=jax.ShapeDtypeStruct((TOK, H), jnp.float32),
        in_specs=[pl.BlockSpec(memory_space=pltpu.VMEM)] * 6,
        out_specs=pl.BlockSpec(memory_space=pltpu.VMEM),
        scratch_shapes=[
            pltpu.VMEM((N_DEV, C, PW), jnp.bfloat16),
            pltpu.VMEM((N_DEV, C, PW), jnp.bfloat16),
            pltpu.VMEM((N_DEV, C, H), jnp.bfloat16),
            pltpu.VMEM((N_DEV, C, H), jnp.bfloat16),
            pltpu.SemaphoreType.DMA((N_DEV,)),
            pltpu.SemaphoreType.DMA((N_DEV,)),
            pltpu.SemaphoreType.DMA((N_DEV,)),
            pltpu.SemaphoreType.DMA((N_DEV,)),
        ],
        compiler_params=pltpu.CompilerParams(collective_id=0),
    )(x, router_W, route_idx, route_row,
      expert_W.astype(jnp.bfloat16), shared_W.astype(jnp.bfloat16))


# device time: 118809 ns/iter; 11.8826x vs baseline; 1.1042x over previous
import jax
import jax.numpy as jnp
from jax import lax
from jax.experimental import pallas as pl
from jax.experimental.pallas import tpu as pltpu

N_DEV = 32
E_LOC = 4
E_TOT = 128
TOK = 1024
D = 512
H = 1024
C = 64
PW = D + 128
S = N_DEV * C
NG = 4
GJ = N_DEV // NG
GR = GJ * C


def kernel(x, router_W, route_idx, expert_W, shared_W):
    route_row = route_idx.reshape(1, TOK)

    def body(x_ref, rW_ref, idx_ref, idxr_ref, eW_ref, sW_ref, outs_ref,
             sendbuf, recvbuf, resbuf, resrec, ssem1, rsem1, ssem2, rsem2):
        my = lax.axis_index("i")

        barrier = pltpu.get_barrier_semaphore()
        for j in range(1, N_DEV):
            pl.semaphore_signal(barrier, inc=1,
                                device_id=(lax.rem(my + j, N_DEV),),
                                device_id_type=pl.DeviceIdType.MESH,)
        pl.semaphore_wait(barrier, N_DEV - 1)

        x32 = x_ref[:, :]
        scores = jnp.dot(x32, rW_ref[:, :], preferred_element_type=jnp.float32)
        smax = jnp.max(scores, axis=-1, keepdims=True)
        pr = jnp.exp(scores - smax)
        pr = pr / jnp.sum(pr, axis=-1, keepdims=True)
        rt_c = idx_ref[:, :]
        rt_r = idxr_ref[:, :]
        onehot = lax.broadcasted_iota(jnp.int32, (TOK, E_TOT), 1) == rt_c
        selp = jnp.sum(jnp.where(onehot, pr, 0.0), axis=1, keepdims=True)
        xs = (x32 * selp).astype(jnp.bfloat16)

        dst_c, dst_r = rt_c // E_LOC, rt_r // E_LOC
        jd_c = lax.rem(dst_c - my + N_DEV, N_DEV)
        jd_r = lax.rem(dst_r - my + N_DEV, N_DEV)
        eloc_c = (rt_c % E_LOC).astype(jnp.bfloat16)
        i0 = lax.broadcasted_iota(jnp.int32, (TOK, TOK), 0)
        i1 = lax.broadcasted_iota(jnp.int32, (TOK, TOK), 1)
        same = (dst_c == dst_r)
        pos_c = jnp.sum((same & (i1 < i0)).astype(jnp.int32),
                        axis=1, keepdims=True)
        pos_r = jnp.sum((same & (i0 < i1)).astype(jnp.int32),
                        axis=0, keepdims=True)
        slot_c = jd_c * C + jnp.minimum(pos_c, C - 1)
        slot_r = jd_r * C + jnp.minimum(pos_r, C - 1)

        payload = jnp.concatenate(
            [xs, jnp.broadcast_to(eloc_c, (TOK, 128))], axis=1)

        disp = []
        for g in range(NG):
            s2_g = (lax.broadcasted_iota(jnp.int32, (GR, TOK), 0) + g * GR
                    == slot_r).astype(jnp.bfloat16)
            sendbuf[g * GJ:(g + 1) * GJ, :, :] = jnp.dot(
                s2_g, payload, preferred_element_type=jnp.float32
            ).astype(jnp.bfloat16).reshape(GJ, C, PW)
            for j in range(g * GJ, (g + 1) * GJ):
                rdma = pltpu.make_async_remote_copy(
                    src_ref=sendbuf.at[j],
                    dst_ref=recvbuf.at[j],
                    send_sem=ssem1.at[j],
                    recv_sem=rsem1.at[j],
                    device_id=(lax.rem(my + j, N_DEV),),
                    device_id_type=pl.DeviceIdType.MESH,
                )
                rdma.start()
                disp.append(rdma)

        outs_ref[:, :] = jnp.dot(
            x32.astype(jnp.bfloat16), sW_ref[:, :],
            preferred_element_type=jnp.float32)

        comb = []
        for g in range(NG):
            for j in range(g * GJ, (g + 1) * GJ):
                disp[j].wait()
            val = recvbuf[g * GJ:(g + 1) * GJ, :, :]
            xin = val[:, :, :D].reshape(GR, D)
            el = val[:, :, D:D + 1].reshape(GR, 1)
            contrib = jnp.zeros((GR, H), jnp.float32)
            for k in range(E_LOC):
                xm = jnp.where(el == jnp.bfloat16(k), xin, jnp.bfloat16(0.0))
                contrib = contrib + jnp.dot(
                    xm, eW_ref[k, :, :], preferred_element_type=jnp.float32)
            resbuf[g * GJ:(g + 1) * GJ, :, :] = (
                contrib.astype(jnp.bfloat16).reshape(GJ, C, H))
            for j in range(g * GJ, (g + 1) * GJ):
                rdma = pltpu.make_async_remote_copy(
                    src_ref=resbuf.at[j],
                    dst_ref=resrec.at[j],
                    send_sem=ssem2.at[j],
                    recv_sem=rsem2.at[j],
                    device_id=(lax.rem(my - j + N_DEV, N_DEV),),
                    device_id_type=pl.DeviceIdType.MESH,
                )
                rdma.start()
                comb.append(rdma)
        for j in range(N_DEV):
            comb[j].wait()

        p_g = (slot_c == lax.broadcasted_iota(jnp.int32, (TOK, S), 1)
               ).astype(jnp.bfloat16)
        expert_out = jnp.dot(p_g, resrec[:, :, :].reshape(S, H),
                             preferred_element_type=jnp.float32)
        outs_ref[:, :] = outs_ref[:, :] + expert_out

    return pl.pallas_call(
        body,
        out_shape=jax.ShapeDtypeStruct((TOK, H), jnp.float32),
        in_specs=[pl.BlockSpec(memory_space=pltpu.VMEM)] * 6,
        out_specs=pl.BlockSpec(memory_space=pltpu.VMEM),
        scratch_shapes=[
            pltpu.VMEM((N_DEV, C, PW), jnp.bfloat16),
            pltpu.VMEM((N_DEV, C, PW), jnp.bfloat16),
            pltpu.VMEM((N_DEV, C, H), jnp.bfloat16),
            pltpu.VMEM((N_DEV, C, H), jnp.bfloat16),
            pltpu.SemaphoreType.DMA((N_DEV,)),
            pltpu.SemaphoreType.DMA((N_DEV,)),
            pltpu.SemaphoreType.DMA((N_DEV,)),
            pltpu.SemaphoreType.DMA((N_DEV,)),
        ],
        compiler_params=pltpu.CompilerParams(collective_id=0),
    )(x, router_W, route_idx, route_row,
      expert_W.astype(jnp.bfloat16), shared_W.astype(jnp.bfloat16))
